# baseline (speedup 1.0000x reference)
import functools

import jax
import jax.numpy as jnp
from jax import lax
from jax.experimental import pallas as pl
from jax.experimental.pallas import tpu as pltpu
from jax.experimental.pallas import tpu_sc as plsc

_NC = 2
_NS = 16
_NW = _NC * _NS
_L = 16


def _make_sc_sampler(P, C, H, W, HWp, OPB, CH):
    assert P % (_NW * CH) == 0
    PW = P // _NW
    NCHUNK = PW // CH
    assert OPB % PW == 0 or PW % OPB == 0
    mesh = plsc.VectorSubcoreMesh(core_axis_name="c", subcore_axis_name="s")

    @functools.partial(
        pl.kernel,
        out_type=jax.ShapeDtypeStruct((P, C), jnp.float32),
        mesh=mesh,
        compiler_params=pltpu.CompilerParams(
            use_tc_tiling_on_sc=False, needs_layout_passes=False),
        scratch_types=[
            pltpu.VMEM((CH,), jnp.float32),
            pltpu.VMEM((CH,), jnp.float32),
            pltpu.VMEM((CH,), jnp.float32),
            pltpu.VMEM((CH,), jnp.float32),
            pltpu.VMEM((CH,), jnp.float32),
            pltpu.VMEM((CH,), jnp.float32),
            pltpu.VMEM((CH,), jnp.int32),
            pltpu.VMEM((CH,), jnp.int32),
            pltpu.VMEM((CH,), jnp.int32),
            pltpu.VMEM((CH,), jnp.int32),
            pltpu.VMEM((CH, C), jnp.float32),
            pltpu.VMEM((CH, C), jnp.float32),
            pltpu.VMEM((CH, C), jnp.float32),
            pltpu.VMEM((CH, C), jnp.float32),
            pltpu.VMEM((CH, C), jnp.float32),
            pltpu.SemaphoreType.DMA,
        ],
    )
    def sampler(table_hbm, gx_hbm, gy_hbm, out_hbm,
                gx_v, gy_v, wa_v, wb_v, wc_v, wd_v, ia_v, ib_v, ic_v, id_v,
                ra_v, rb_v, rc_v, rd_v, out_v, sem):
        wid = lax.axis_index("s") * _NC + lax.axis_index("c")

        def chunk_body(t, carry):
            base = wid * PW + t * CH
            pltpu.sync_copy(gx_hbm.at[pl.ds(base, CH)], gx_v)
            pltpu.sync_copy(gy_hbm.at[pl.ds(base, CH)], gy_v)
            nbase = (base // OPB) * HWp

            for j in range(CH // _L):
                sl = pl.ds(j * _L, _L)
                gxv = gx_v[sl]
                gyv = gy_v[sl]
                x = (gxv + 1.0) * ((W - 1) * 0.5)
                y = (gyv + 1.0) * ((H - 1) * 0.5)
                xt = x.astype(jnp.int32)
                x0 = jnp.where(xt.astype(jnp.float32) > x, xt - 1, xt)
                yt = y.astype(jnp.int32)
                y0 = jnp.where(yt.astype(jnp.float32) > y, yt - 1, yt)
                fx = x - x0.astype(jnp.float32)
                fy = y - y0.astype(jnp.float32)
                x1 = x0 + 1
                y1 = y0 + 1
                vx0 = (x0 >= 0) & (x0 < W)
                vx1 = (x1 >= 0) & (x1 < W)
                vy0 = (y0 >= 0) & (y0 < H)
                vy1 = (y1 >= 0) & (y1 < H)
                gx1 = 1.0 - fx
                gy1 = 1.0 - fy
                zero = jnp.zeros_like(fx)
                wa_v[sl] = jnp.where(vx0 & vy0, gx1 * gy1, zero)
                wb_v[sl] = jnp.where(vx0 & vy1, gx1 * fy, zero)
                wc_v[sl] = jnp.where(vx1 & vy0, fx * gy1, zero)
                wd_v[sl] = jnp.where(vx1 & vy1, fx * fy, zero)
                xc0 = jnp.clip(x0, 0, W - 1)
                xc1 = jnp.clip(x1, 0, W - 1)
                yc0 = jnp.clip(y0, 0, H - 1)
                yc1 = jnp.clip(y1, 0, H - 1)
                r0 = nbase + yc0 * W
                r1 = nbase + yc1 * W
                ia_v[sl] = r0 + xc0
                ib_v[sl] = r1 + xc0
                ic_v[sl] = r0 + xc1
                id_v[sl] = r1 + xc1

            ha = pltpu.async_copy(table_hbm.at[ia_v], ra_v, sem)
            hb = pltpu.async_copy(table_hbm.at[ib_v], rb_v, sem)
            hc = pltpu.async_copy(table_hbm.at[ic_v], rc_v, sem)
            hd = pltpu.async_copy(table_hbm.at[id_v], rd_v, sem)
            ha.wait()
            hb.wait()
            hc.wait()
            hd.wait()

            def row_body(r, carry2):
                ridx = jnp.full((_L,), 0, jnp.int32) + r
                war = plsc.load_gather(wa_v, [ridx])
                wbr = plsc.load_gather(wb_v, [ridx])
                wcr = plsc.load_gather(wc_v, [ridx])
                wdr = plsc.load_gather(wd_v, [ridx])
                for k in range(C // _L):
                    s2 = pl.ds(k * _L, _L)
                    acc = war * ra_v[r, s2]
                    acc = acc + wbr * rb_v[r, s2]
                    acc = acc + wcr * rc_v[r, s2]
                    acc = acc + wdr * rd_v[r, s2]
                    out_v[r, s2] = acc
                return carry2

            lax.fori_loop(0, CH, row_body, 0)
            pltpu.sync_copy(out_v, out_hbm.at[pl.ds(base, CH)])
            return carry

        lax.fori_loop(0, NCHUNK, chunk_body, 0)

    return sampler


def kernel(tenInput, g):
    N, C, H, W = tenInput.shape
    Ho, Wo = g.shape[1], g.shape[2]
    P = N * Ho * Wo
    table = tenInput.transpose(0, 2, 3, 1).reshape(N * H * W, C)
    gx = g[..., 0].reshape(P)
    gy = g[..., 1].reshape(P)
    sampler = _make_sc_sampler(P, C, H, W, H * W, Ho * Wo, 128)
    out_flat = sampler(table, gx, gy)
    return out_flat.reshape(N, Ho, Wo, C).transpose(0, 3, 1, 2)


# speedup vs baseline: 1.3878x; 1.3878x over previous
import functools

import jax
import jax.numpy as jnp
from jax import lax
from jax.experimental import pallas as pl
from jax.experimental.pallas import tpu as pltpu
from jax.experimental.pallas import tpu_sc as plsc

_NC = 2
_NS = 16
_NW = _NC * _NS
_L = 16


def _make_sc_sampler(P, C, CP, H, W, HWp, OPB, CH):
    assert P % (_NW * CH) == 0
    PW = P // _NW
    NCH = PW // CH
    assert NCH % 2 == 0
    assert OPB % PW == 0 or PW % OPB == 0
    mesh = plsc.VectorSubcoreMesh(core_axis_name="c", subcore_axis_name="s")

    rows_t = pltpu.VMEM((CH, CP), jnp.float32)
    wvec_t = pltpu.VMEM((CH,), jnp.float32)
    ivec_t = pltpu.VMEM((CH,), jnp.int32)

    @functools.partial(
        pl.kernel,
        out_type=jax.ShapeDtypeStruct((P, CP), jnp.float32),
        mesh=mesh,
        compiler_params=pltpu.CompilerParams(needs_layout_passes=False),
        scratch_types=[
            pltpu.VMEM((CH,), jnp.float32),
            pltpu.VMEM((CH,), jnp.float32),
            [wvec_t] * 8,
            [ivec_t] * 8,
            [rows_t] * 8,
            [pltpu.VMEM((CH, CP), jnp.float32)] * 2,
            [pltpu.SemaphoreType.DMA] * 4,
        ],
    )
    def sampler(table_hbm, gx_hbm, gy_hbm, out_hbm, gx_v, gy_v, w8, i8, r8, outs, sems):
        wid = lax.axis_index("s") * _NC + lax.axis_index("c")
        wsets = (w8[0:4], w8[4:8])
        isets = (i8[0:4], i8[4:8])
        rsets = (r8[0:4], r8[4:8])
        gsems = (sems[0], sems[1])
        osems = (sems[2], sems[3])

        def chunk_base(i):
            return wid * PW + i * CH

        def prep(i, s):
            base = chunk_base(i)
            pltpu.sync_copy(gx_hbm.at[pl.ds(base, CH)], gx_v)
            pltpu.sync_copy(gy_hbm.at[pl.ds(base, CH)], gy_v)
            nbase = (base // OPB) * HWp
            wa_v, wb_v, wc_v, wd_v = wsets[s]
            ia_v, ib_v, ic_v, id_v = isets[s]
            for j in range(CH // _L):
                sl = pl.ds(j * _L, _L)
                x = (gx_v[sl] + 1.0) * ((W - 1) * 0.5)
                y = (gy_v[sl] + 1.0) * ((H - 1) * 0.5)
                xt = x.astype(jnp.int32)
                x0 = jnp.where(xt.astype(jnp.float32) > x, xt - 1, xt)
                yt = y.astype(jnp.int32)
                y0 = jnp.where(yt.astype(jnp.float32) > y, yt - 1, yt)
                fx = x - x0.astype(jnp.float32)
                fy = y - y0.astype(jnp.float32)
                x1 = x0 + 1
                y1 = y0 + 1
                vx0 = (x0 >= 0) & (x0 < W)
                vx1 = (x1 >= 0) & (x1 < W)
                vy0 = (y0 >= 0) & (y0 < H)
                vy1 = (y1 >= 0) & (y1 < H)
                gx1 = 1.0 - fx
                gy1 = 1.0 - fy
                zero = jnp.zeros_like(fx)
                wa_v[sl] = jnp.where(vx0 & vy0, gx1 * gy1, zero)
                wb_v[sl] = jnp.where(vx0 & vy1, gx1 * fy, zero)
                wc_v[sl] = jnp.where(vx1 & vy0, fx * gy1, zero)
                wd_v[sl] = jnp.where(vx1 & vy1, fx * fy, zero)
                xc0 = jnp.clip(x0, 0, W - 1)
                xc1 = jnp.clip(x1, 0, W - 1)
                r0 = nbase + jnp.clip(y0, 0, H - 1) * W
                r1 = nbase + jnp.clip(y1, 0, H - 1) * W
                ia_v[sl] = r0 + xc0
                ib_v[sl] = r1 + xc0
                ic_v[sl] = r0 + xc1
                id_v[sl] = r1 + xc1

        def fire_gathers(s):
            for iv, rv in zip(isets[s], rsets[s]):
                pltpu.async_copy(table_hbm.at[iv], rv, gsems[s])

        def wait_gathers(s):
            for iv, rv in zip(isets[s], rsets[s]):
                pltpu.make_async_copy(table_hbm.at[iv], rv, gsems[s]).wait()

        def accumulate(s):
            wa_v, wb_v, wc_v, wd_v = wsets[s]
            ra_v, rb_v, rc_v, rd_v = rsets[s]
            out_v = outs[s]

            def row_body(r, carry):
                ridx = jnp.full((_L,), 0, jnp.int32) + r
                war = plsc.load_gather(wa_v, [ridx])
                wbr = plsc.load_gather(wb_v, [ridx])
                wcr = plsc.load_gather(wc_v, [ridx])
                wdr = plsc.load_gather(wd_v, [ridx])
                for k in range(C // _L):
                    s2 = pl.ds(k * _L, _L)
                    acc = war * ra_v[r, s2]
                    acc = acc + wbr * rb_v[r, s2]
                    acc = acc + wcr * rc_v[r, s2]
                    acc = acc + wdr * rd_v[r, s2]
                    out_v[r, s2] = acc
                return carry

            lax.fori_loop(0, CH, row_body, 0)

        def fire_store(i, s):
            pltpu.async_copy(outs[s], out_hbm.at[pl.ds(chunk_base(i), CH)],
                             osems[s])

        def wait_store(i, s):
            pltpu.make_async_copy(outs[s], out_hbm.at[pl.ds(chunk_base(i), CH)],
                                  osems[s]).wait()

        prep(0, 0)
        fire_gathers(0)

        def pair_body(tt, carry):
            i0 = 2 * tt
            prep(i0 + 1, 1)
            fire_gathers(1)
            wait_gathers(0)

            @pl.when(tt > 0)
            def _():
                wait_store(i0, 0)

            accumulate(0)
            fire_store(i0, 0)

            @pl.when(i0 + 2 < NCH)
            def _():
                prep(i0 + 2, 0)
                fire_gathers(0)

            wait_gathers(1)

            @pl.when(tt > 0)
            def _():
                wait_store(i0 + 1, 1)

            accumulate(1)
            fire_store(i0 + 1, 1)
            return carry

        lax.fori_loop(0, NCH // 2, pair_body, 0)
        wait_store(NCH - 2, 0)
        wait_store(NCH - 1, 1)

    return sampler


def kernel(tenInput, g):
    N, C, H, W = tenInput.shape
    Ho, Wo = g.shape[1], g.shape[2]
    P = N * Ho * Wo
    CP = 128
    tin = tenInput.transpose(0, 2, 3, 1)
    table = jnp.pad(tin, ((0, 0), (0, 0), (0, 0), (0, CP - C)))
    table = table.reshape(N * H * W, CP)
    gx = g[..., 0].reshape(P)
    gy = g[..., 1].reshape(P)
    sampler = _make_sc_sampler(P, C, CP, H, W, H * W, Ho * Wo, 64)
    out_flat = sampler(table, gx, gy)
    return out_flat.reshape(N, Ho, Wo, CP)[..., :C].transpose(0, 3, 1, 2)
